# Initial kernel scaffold; baseline (speedup 1.0000x reference)
import jax
import jax.numpy as jnp
from jax.experimental import pallas as pl


def kernel(x, edge_index, W1, b1, W2l, b2, W2r, W3l, b3, W3r, W4l, b4, W4r, Wm1, bm1, Wm2, bm2):
    raise NotImplementedError("write your pallas kernel here")


# speedup vs baseline: 8.5919x; 8.5919x over previous
import functools

import jax
import jax.numpy as jnp
from jax import lax
from jax.experimental import pallas as pl
from jax.experimental.pallas import tpu as pltpu
from jax.experimental.pallas import tpu_sc as plsc

N = 50000
E = 800000
NP = 50176
NC = 2
NS = 16
NW = NC * NS
B = 128
NB = 200
KB = 8
RB = 4
NSUP = NB // KB
EP = NW * NB * B
RPT = NP // NS


def _sc_spmm(src3, dst3, zeros32, tables):
    C = len(tables)
    mesh = plsc.VectorSubcoreMesh(core_axis_name="c", subcore_axis_name="s")

    def body(src_ref, dst_ref, z_ref, *rest):
        table_refs = rest[:C]
        out_ref = rest[C]
        acc, idx_s, idx_d, rows, sem = rest[C + 1:]
        cid = lax.axis_index("c")
        sid = lax.axis_index("s")
        w = cid * NS + sid
        base = sid * RPT

        def zero_mine():
            pltpu.sync_copy(z_ref.at[pl.ds(base, RPT)],
                            acc.at[pl.ds(base, RPT)])

        zero_mine()
        plsc.subcore_barrier()
        for c in range(C):
            t_ref = table_refs[c]

            def sup(i, carry):
                pltpu.sync_copy(src_ref.at[w, pl.ds(i * KB, KB)], idx_s)
                pltpu.sync_copy(dst_ref.at[w, pl.ds(i * KB, KB)], idx_d)
                for h in range(KB // RB):
                    descs = [pltpu.async_copy(
                        t_ref.at[idx_s.at[RB * h + j]], rows.at[j], sem)
                        for j in range(RB)]
                    for d in descs:
                        d.wait()
                    for j in range(RB):
                        pltpu.sync_copy(rows.at[j],
                                        acc.at[idx_d.at[RB * h + j]],
                                        add=True)
                return carry

            lax.fori_loop(0, NSUP, sup, 0)
            plsc.subcore_barrier()
            pltpu.sync_copy(acc.at[pl.ds(base, RPT)],
                            out_ref.at[cid, c, pl.ds(base, RPT)])
            if c < C - 1:
                zero_mine()
                plsc.subcore_barrier()

    f = pl.kernel(
        body,
        out_type=jax.ShapeDtypeStruct((NC, C, NP, 32), jnp.float32),
        mesh=mesh,
        compiler_params=pltpu.CompilerParams(use_tc_tiling_on_sc=False),
        scratch_types=[
            pltpu.VMEM_SHARED((NP, 32), jnp.float32),
            pltpu.VMEM((KB, B), jnp.int32),
            pltpu.VMEM((KB, B), jnp.int32),
            pltpu.VMEM((RB, B, 32), jnp.float32),
            pltpu.SemaphoreType.DMA,
        ],
    )
    return f(src3, dst3, zeros32, *tables)


def _sc_hist(dst3, zeros16, ones16):
    mesh = plsc.VectorSubcoreMesh(core_axis_name="c", subcore_axis_name="s")

    def body(dst_ref, z_ref, ones_ref, out_ref, acc, idx_d, ones_v, sem):
        cid = lax.axis_index("c")
        sid = lax.axis_index("s")
        w = cid * NS + sid
        base = sid * RPT
        pltpu.sync_copy(ones_ref, ones_v)
        pltpu.sync_copy(z_ref.at[pl.ds(base, RPT)], acc.at[pl.ds(base, RPT)])
        plsc.subcore_barrier()

        def sup(i, carry):
            pltpu.sync_copy(dst_ref.at[w, pl.ds(i * KB, KB)], idx_d)
            for j in range(KB):
                pltpu.sync_copy(ones_v, acc.at[idx_d.at[j]], add=True)
            return carry

        lax.fori_loop(0, NSUP, sup, 0)
        plsc.subcore_barrier()
        pltpu.sync_copy(acc.at[pl.ds(base, RPT)],
                        out_ref.at[cid, pl.ds(base, RPT)])

    f = pl.kernel(
        body,
        out_type=jax.ShapeDtypeStruct((NC, NP, 16), jnp.float32),
        mesh=mesh,
        compiler_params=pltpu.CompilerParams(use_tc_tiling_on_sc=False),
        scratch_types=[
            pltpu.VMEM_SHARED((NP, 16), jnp.float32),
            pltpu.VMEM((KB, B), jnp.int32),
            pltpu.VMEM((B, 16), jnp.float32),
            pltpu.SemaphoreType.DMA,
        ],
    )
    return f(dst3, zeros16, ones16)


GRID = 32
BN = NP // GRID

_node = lambda d: pl.BlockSpec((BN, d), lambda i: (i, 0))
_hist = pl.BlockSpec((NC, BN, 16), lambda i: (0, i, 0))
_part = lambda c: pl.BlockSpec((NC, c, BN, 32), lambda i: (0, 0, i, 0))
_full = lambda a, b: pl.BlockSpec((a, b), lambda i: (0, 0))


def _cnt_dinv(hp_ref):
    cnt = hp_ref[0][:, 0:1] + hp_ref[1][:, 0:1]
    return jnp.maximum(cnt, 1.0), lax.rsqrt(cnt + 1.0)


def _tc1(hp, x8, w1p):
    def body(hp_ref, x_ref, w_ref, o_ref):
        _, dinv = _cnt_dinv(hp_ref)
        o_ref[...] = jnp.dot(x_ref[...], w_ref[...],
                             preferred_element_type=jnp.float32) * dinv

    return pl.pallas_call(
        body, grid=(GRID,),
        in_specs=[_hist, _node(8), _full(8, 32)],
        out_specs=_node(32),
        out_shape=jax.ShapeDtypeStruct((NP, 32), jnp.float32),
    )(hp, x8, w1p)


def _tc2(p1, hp, g1, b1r):
    def body(p_ref, hp_ref, g_ref, b_ref, o_ref):
        _, dinv = _cnt_dinv(hp_ref)
        s = p_ref[0, 0] + p_ref[1, 0] + g_ref[...]
        o_ref[...] = jnp.maximum(s * dinv + b_ref[...], 0.0)

    return pl.pallas_call(
        body, grid=(GRID,),
        in_specs=[_part(1), _hist, _node(32), _full(1, 32)],
        out_specs=_node(32),
        out_shape=jax.ShapeDtypeStruct((NP, 32), jnp.float32),
    )(p1, hp, g1, b1r)


def _tc3(p2, hp, h1, w2l, b2r, w2r):
    def body(p_ref, hp_ref, h_ref, wl_ref, b_ref, wr_ref, *o_refs):
        cnt1, _ = _cnt_dinv(hp_ref)
        agg = (p_ref[0, 0] + p_ref[1, 0]) / cnt1
        h2 = jnp.dot(agg, wl_ref[...], preferred_element_type=jnp.float32)
        h2 = h2 + b_ref[...] + jnp.dot(h_ref[...], wr_ref[...],
                                       preferred_element_type=jnp.float32)
        h2 = jnp.maximum(h2, 0.0)
        for c in range(4):
            o_refs[c][...] = h2[:, 32 * c:32 * (c + 1)]

    shp = jax.ShapeDtypeStruct((NP, 32), jnp.float32)
    return pl.pallas_call(
        body, grid=(GRID,),
        in_specs=[_part(1), _hist, _node(32), _full(32, 128), _full(1, 128),
                  _full(32, 128)],
        out_specs=[_node(32)] * 4,
        out_shape=[shp] * 4,
    )(p2, hp, h1, w2l, b2r, w2r)


def _tc4(p3, hp, h2c, w3l, b3r, w3r, w4l):
    def body(p_ref, hp_ref, hc0, hc1, hc2, hc3, wl_ref, b_ref, wr_ref,
             w4_ref, *o_refs):
        cnt1, _ = _cnt_dinv(hp_ref)
        hcs = (hc0, hc1, hc2, hc3)
        h3 = b_ref[...]
        for c in range(4):
            agg = (p_ref[0, c] + p_ref[1, c]) / cnt1
            h3 = h3 + jnp.dot(agg, wl_ref[32 * c:32 * (c + 1), :],
                              preferred_element_type=jnp.float32)
            h3 = h3 + jnp.dot(hcs[c][...], wr_ref[32 * c:32 * (c + 1), :],
                              preferred_element_type=jnp.float32)
        h3 = jnp.maximum(h3, 0.0)
        for c in range(4):
            o_refs[c][...] = h3[:, 32 * c:32 * (c + 1)]
        o_refs[4][...] = jnp.dot(h3, w4_ref[...],
                                 preferred_element_type=jnp.float32)

    shp = jax.ShapeDtypeStruct((NP, 32), jnp.float32)
    return pl.pallas_call(
        body, grid=(GRID,),
        in_specs=[_part(4), _hist] + [_node(32)] * 4 +
                 [_full(128, 128), _full(1, 128), _full(128, 128),
                  _full(128, 32)],
        out_specs=[_node(32)] * 5,
        out_shape=[shp] * 5,
    )(p3, hp, *h2c, w3l, b3r, w3r, w4l)


def _tc5(p4, hp, h3c, w4r, b4r, wm1, bm1r, wm2p, bm2p):
    def body(p_ref, hp_ref, hc0, hc1, hc2, hc3, wr_ref, b_ref, w1_ref,
             b1_ref, w2_ref, b2_ref, o_ref):
        cnt1, _ = _cnt_dinv(hp_ref)
        hcs = (hc0, hc1, hc2, hc3)
        h4 = (p_ref[0, 0] + p_ref[1, 0]) / cnt1 + b_ref[...]
        for c in range(4):
            h4 = h4 + jnp.dot(hcs[c][...], wr_ref[32 * c:32 * (c + 1), :],
                              preferred_element_type=jnp.float32)
        m = jnp.maximum(jnp.dot(h4, w1_ref[...],
                                preferred_element_type=jnp.float32)
                        + b1_ref[...], 0.0)
        y = jnp.dot(m, w2_ref[...], preferred_element_type=jnp.float32)
        o_ref[...] = jax.nn.sigmoid(y + b2_ref[...])

    return pl.pallas_call(
        body, grid=(GRID,),
        in_specs=[_part(1), _hist] + [_node(32)] * 4 +
                 [_full(128, 32), _full(1, 32), _full(32, 16), _full(1, 16),
                  _full(16, 8), _full(1, 8)],
        out_specs=_node(8),
        out_shape=jax.ShapeDtypeStruct((NP, 8), jnp.float32),
    )(p4, hp, *h3c, w4r, b4r, wm1, bm1r, wm2p, bm2p)


def kernel(x, edge_index, W1, b1, W2l, b2, W2r, W3l, b3, W3r, W4l, b4, W4r,
           Wm1, bm1, Wm2, bm2):
    src = jnp.concatenate([edge_index[0],
                           jnp.zeros((EP - E,), jnp.int32)]).reshape(
                               NW, NB, B)
    dst = jnp.concatenate([edge_index[1],
                           jnp.full((EP - E,), N, jnp.int32)]).reshape(
                               NW, NB, B)
    zeros32 = jnp.zeros((NP, 32), jnp.float32)
    zeros16 = jnp.zeros((NP, 16), jnp.float32)
    ones16 = jnp.ones((B, 16), jnp.float32)
    x8 = jnp.pad(x, ((0, NP - N), (0, 8 - x.shape[1])))
    w1p = jnp.pad(W1, ((0, 8 - W1.shape[0]), (0, 0)))
    wm2p = jnp.pad(Wm2, ((0, 0), (0, 8 - Wm2.shape[1])))
    bm2p = jnp.pad(bm2, (0, 8 - bm2.shape[0])).reshape(1, 8)
    b1r, b2r, b3r, b4r = (b.reshape(1, -1) for b in (b1, b2, b3, b4))
    bm1r = bm1.reshape(1, -1)

    hp = _sc_hist(dst, zeros16, ones16)
    g1 = _tc1(hp, x8, w1p)
    p1 = _sc_spmm(src, dst, zeros32, [g1])
    h1 = _tc2(p1, hp, g1, b1r)
    p2 = _sc_spmm(src, dst, zeros32, [h1])
    h2c = _tc3(p2, hp, h1, W2l, b2r, W2r)
    p3 = _sc_spmm(src, dst, zeros32, list(h2c))
    *h3c, g4 = _tc4(p3, hp, h2c, W3l, b3r, W3r, W4l)
    p4 = _sc_spmm(src, dst, zeros32, [g4])
    y = _tc5(p4, hp, h3c, W4r, b4r, Wm1, bm1r, wm2p, bm2p)
    return y[:N, 0]


# speedup vs baseline: 9.8076x; 1.1415x over previous
import functools

import jax
import jax.numpy as jnp
from jax import lax
from jax.experimental import pallas as pl
from jax.experimental.pallas import tpu as pltpu
from jax.experimental.pallas import tpu_sc as plsc

N = 50000
E = 800000
NP = 50176
NC = 2
NS = 16
NW = NC * NS
B = 128
NB = 200
KB = 8
RB = 4
NSUP = NB // KB
EP = NW * NB * B
RPT = NP // NS


def _sc_spmm(src3, dst3, zeros32, tables):
    C = len(tables)
    mesh = plsc.VectorSubcoreMesh(core_axis_name="c", subcore_axis_name="s")

    def body(src_ref, dst_ref, z_ref, *rest):
        table_refs = rest[:C]
        out_ref = rest[C]
        acc, idx_s, idx_d, rows = rest[C + 1:C + 5]
        gsems = rest[C + 5:C + 5 + RB]
        isem = rest[C + 5 + RB]
        cid = lax.axis_index("c")
        sid = lax.axis_index("s")
        w = cid * NS + sid
        base = sid * RPT

        def zero_mine():
            pltpu.sync_copy(z_ref.at[pl.ds(base, RPT)],
                            acc.at[pl.ds(base, RPT)])

        def idx_fetch(i, sl):
            pltpu.async_copy(src_ref.at[w, pl.ds(i * KB, KB)],
                             idx_s.at[sl], isem)
            pltpu.async_copy(dst_ref.at[w, pl.ds(i * KB, KB)],
                             idx_d.at[sl], isem)

        zero_mine()
        plsc.subcore_barrier()
        for c in range(C):
            t_ref = table_refs[c]
            idx_fetch(0, 0)

            def sup(i, carry):
                sl = lax.rem(i, 2)
                pltpu.make_async_copy(src_ref.at[w, pl.ds(i * KB, KB)],
                                      idx_s.at[sl], isem).wait()
                pltpu.make_async_copy(dst_ref.at[w, pl.ds(i * KB, KB)],
                                      idx_d.at[sl], isem).wait()

                @pl.when(i + 1 < NSUP)
                def _():
                    idx_fetch(i + 1, 1 - sl)

                for j in range(RB):
                    pltpu.async_copy(t_ref.at[idx_s.at[sl, j]], rows.at[j],
                                     gsems[j])
                for j in range(KB):
                    q = j % RB
                    pltpu.make_async_copy(t_ref.at[idx_s.at[sl, j]],
                                          rows.at[q], gsems[q]).wait()
                    pltpu.sync_copy(rows.at[q], acc.at[idx_d.at[sl, j]],
                                    add=True)
                    if j + RB < KB:
                        pltpu.async_copy(t_ref.at[idx_s.at[sl, j + RB]],
                                         rows.at[q], gsems[q])
                return carry

            lax.fori_loop(0, NSUP, sup, 0)
            plsc.subcore_barrier()
            pltpu.sync_copy(acc.at[pl.ds(base, RPT)],
                            out_ref.at[cid, c, pl.ds(base, RPT)])
            if c < C - 1:
                zero_mine()
                plsc.subcore_barrier()

    f = pl.kernel(
        body,
        out_type=jax.ShapeDtypeStruct((NC, C, NP, 32), jnp.float32),
        mesh=mesh,
        compiler_params=pltpu.CompilerParams(use_tc_tiling_on_sc=False),
        scratch_types=[
            pltpu.VMEM_SHARED((NP, 32), jnp.float32),
            pltpu.VMEM((2, KB, B), jnp.int32),
            pltpu.VMEM((2, KB, B), jnp.int32),
            pltpu.VMEM((RB, B, 32), jnp.float32),
        ] + [pltpu.SemaphoreType.DMA] * (RB + 1),
    )
    return f(src3, dst3, zeros32, *tables)


def _sc_hist(dst3, zeros16, ones16):
    mesh = plsc.VectorSubcoreMesh(core_axis_name="c", subcore_axis_name="s")

    def body(dst_ref, z_ref, ones_ref, out_ref, acc, idx_d, ones_v, sem):
        cid = lax.axis_index("c")
        sid = lax.axis_index("s")
        w = cid * NS + sid
        base = sid * RPT
        pltpu.sync_copy(ones_ref, ones_v)
        pltpu.sync_copy(z_ref.at[pl.ds(base, RPT)], acc.at[pl.ds(base, RPT)])
        plsc.subcore_barrier()

        def sup(i, carry):
            pltpu.sync_copy(dst_ref.at[w, pl.ds(i * KB, KB)], idx_d)
            for j in range(KB):
                pltpu.sync_copy(ones_v, acc.at[idx_d.at[j]], add=True)
            return carry

        lax.fori_loop(0, NSUP, sup, 0)
        plsc.subcore_barrier()
        pltpu.sync_copy(acc.at[pl.ds(base, RPT)],
                        out_ref.at[cid, pl.ds(base, RPT)])

    f = pl.kernel(
        body,
        out_type=jax.ShapeDtypeStruct((NC, NP, 16), jnp.float32),
        mesh=mesh,
        compiler_params=pltpu.CompilerParams(use_tc_tiling_on_sc=False),
        scratch_types=[
            pltpu.VMEM_SHARED((NP, 16), jnp.float32),
            pltpu.VMEM((KB, B), jnp.int32),
            pltpu.VMEM((B, 16), jnp.float32),
            pltpu.SemaphoreType.DMA,
        ],
    )
    return f(dst3, zeros16, ones16)


GRID = 32
BN = NP // GRID

_node = lambda d: pl.BlockSpec((BN, d), lambda i: (i, 0))
_hist = pl.BlockSpec((NC, BN, 16), lambda i: (0, i, 0))
_part = lambda c: pl.BlockSpec((NC, c, BN, 32), lambda i: (0, 0, i, 0))
_full = lambda a, b: pl.BlockSpec((a, b), lambda i: (0, 0))


def _cnt_dinv(hp_ref):
    cnt = hp_ref[0][:, 0:1] + hp_ref[1][:, 0:1]
    return jnp.maximum(cnt, 1.0), lax.rsqrt(cnt + 1.0)


def _tc1(hp, x8, w1p):
    def body(hp_ref, x_ref, w_ref, o_ref):
        _, dinv = _cnt_dinv(hp_ref)
        o_ref[...] = jnp.dot(x_ref[...], w_ref[...],
                             preferred_element_type=jnp.float32) * dinv

    return pl.pallas_call(
        body, grid=(GRID,),
        in_specs=[_hist, _node(8), _full(8, 32)],
        out_specs=_node(32),
        out_shape=jax.ShapeDtypeStruct((NP, 32), jnp.float32),
    )(hp, x8, w1p)


def _tc2(p1, hp, g1, b1r):
    def body(p_ref, hp_ref, g_ref, b_ref, o_ref):
        _, dinv = _cnt_dinv(hp_ref)
        s = p_ref[0, 0] + p_ref[1, 0] + g_ref[...]
        o_ref[...] = jnp.maximum(s * dinv + b_ref[...], 0.0)

    return pl.pallas_call(
        body, grid=(GRID,),
        in_specs=[_part(1), _hist, _node(32), _full(1, 32)],
        out_specs=_node(32),
        out_shape=jax.ShapeDtypeStruct((NP, 32), jnp.float32),
    )(p1, hp, g1, b1r)


def _tc3(p2, hp, h1, w2l, b2r, w2r):
    def body(p_ref, hp_ref, h_ref, wl_ref, b_ref, wr_ref, *o_refs):
        cnt1, _ = _cnt_dinv(hp_ref)
        agg = (p_ref[0, 0] + p_ref[1, 0]) / cnt1
        h2 = jnp.dot(agg, wl_ref[...], preferred_element_type=jnp.float32)
        h2 = h2 + b_ref[...] + jnp.dot(h_ref[...], wr_ref[...],
                                       preferred_element_type=jnp.float32)
        h2 = jnp.maximum(h2, 0.0)
        for c in range(4):
            o_refs[c][...] = h2[:, 32 * c:32 * (c + 1)]

    shp = jax.ShapeDtypeStruct((NP, 32), jnp.float32)
    return pl.pallas_call(
        body, grid=(GRID,),
        in_specs=[_part(1), _hist, _node(32), _full(32, 128), _full(1, 128),
                  _full(32, 128)],
        out_specs=[_node(32)] * 4,
        out_shape=[shp] * 4,
    )(p2, hp, h1, w2l, b2r, w2r)


def _tc4(p3, hp, h2c, w3l, b3r, w3r, w4l):
    def body(p_ref, hp_ref, hc0, hc1, hc2, hc3, wl_ref, b_ref, wr_ref,
             w4_ref, *o_refs):
        cnt1, _ = _cnt_dinv(hp_ref)
        hcs = (hc0, hc1, hc2, hc3)
        h3 = b_ref[...]
        for c in range(4):
            agg = (p_ref[0, c] + p_ref[1, c]) / cnt1
            h3 = h3 + jnp.dot(agg, wl_ref[32 * c:32 * (c + 1), :],
                              preferred_element_type=jnp.float32)
            h3 = h3 + jnp.dot(hcs[c][...], wr_ref[32 * c:32 * (c + 1), :],
                              preferred_element_type=jnp.float32)
        h3 = jnp.maximum(h3, 0.0)
        for c in range(4):
            o_refs[c][...] = h3[:, 32 * c:32 * (c + 1)]
        o_refs[4][...] = jnp.dot(h3, w4_ref[...],
                                 preferred_element_type=jnp.float32)

    shp = jax.ShapeDtypeStruct((NP, 32), jnp.float32)
    return pl.pallas_call(
        body, grid=(GRID,),
        in_specs=[_part(4), _hist] + [_node(32)] * 4 +
                 [_full(128, 128), _full(1, 128), _full(128, 128),
                  _full(128, 32)],
        out_specs=[_node(32)] * 5,
        out_shape=[shp] * 5,
    )(p3, hp, *h2c, w3l, b3r, w3r, w4l)


def _tc5(p4, hp, h3c, w4r, b4r, wm1, bm1r, wm2p, bm2p):
    def body(p_ref, hp_ref, hc0, hc1, hc2, hc3, wr_ref, b_ref, w1_ref,
             b1_ref, w2_ref, b2_ref, o_ref):
        cnt1, _ = _cnt_dinv(hp_ref)
        hcs = (hc0, hc1, hc2, hc3)
        h4 = (p_ref[0, 0] + p_ref[1, 0]) / cnt1 + b_ref[...]
        for c in range(4):
            h4 = h4 + jnp.dot(hcs[c][...], wr_ref[32 * c:32 * (c + 1), :],
                              preferred_element_type=jnp.float32)
        m = jnp.maximum(jnp.dot(h4, w1_ref[...],
                                preferred_element_type=jnp.float32)
                        + b1_ref[...], 0.0)
        y = jnp.dot(m, w2_ref[...], preferred_element_type=jnp.float32)
        o_ref[...] = jax.nn.sigmoid(y + b2_ref[...])

    return pl.pallas_call(
        body, grid=(GRID,),
        in_specs=[_part(1), _hist] + [_node(32)] * 4 +
                 [_full(128, 32), _full(1, 32), _full(32, 16), _full(1, 16),
                  _full(16, 8), _full(1, 8)],
        out_specs=_node(8),
        out_shape=jax.ShapeDtypeStruct((NP, 8), jnp.float32),
    )(p4, hp, *h3c, w4r, b4r, wm1, bm1r, wm2p, bm2p)


def kernel(x, edge_index, W1, b1, W2l, b2, W2r, W3l, b3, W3r, W4l, b4, W4r,
           Wm1, bm1, Wm2, bm2):
    src = jnp.concatenate([edge_index[0],
                           jnp.zeros((EP - E,), jnp.int32)]).reshape(
                               NW, NB, B)
    dst = jnp.concatenate([edge_index[1],
                           jnp.full((EP - E,), N, jnp.int32)]).reshape(
                               NW, NB, B)
    zeros32 = jnp.zeros((NP, 32), jnp.float32)
    zeros16 = jnp.zeros((NP, 16), jnp.float32)
    ones16 = jnp.ones((B, 16), jnp.float32)
    x8 = jnp.pad(x, ((0, NP - N), (0, 8 - x.shape[1])))
    w1p = jnp.pad(W1, ((0, 8 - W1.shape[0]), (0, 0)))
    wm2p = jnp.pad(Wm2, ((0, 0), (0, 8 - Wm2.shape[1])))
    bm2p = jnp.pad(bm2, (0, 8 - bm2.shape[0])).reshape(1, 8)
    b1r, b2r, b3r, b4r = (b.reshape(1, -1) for b in (b1, b2, b3, b4))
    bm1r = bm1.reshape(1, -1)

    hp = _sc_hist(dst, zeros16, ones16)
    g1 = _tc1(hp, x8, w1p)
    p1 = _sc_spmm(src, dst, zeros32, [g1])
    h1 = _tc2(p1, hp, g1, b1r)
    p2 = _sc_spmm(src, dst, zeros32, [h1])
    h2c = _tc3(p2, hp, h1, W2l, b2r, W2r)
    p3 = _sc_spmm(src, dst, zeros32, list(h2c))
    *h3c, g4 = _tc4(p3, hp, h2c, W3l, b3r, W3r, W4l)
    p4 = _sc_spmm(src, dst, zeros32, [g4])
    y = _tc5(p4, hp, h3c, W4r, b4r, Wm1, bm1r, wm2p, bm2p)
    return y[:N, 0]


# speedup vs baseline: 10.1538x; 1.0353x over previous
import functools

import jax
import jax.numpy as jnp
from jax import lax
from jax.experimental import pallas as pl
from jax.experimental.pallas import tpu as pltpu
from jax.experimental.pallas import tpu_sc as plsc

N = 50000
E = 800000
NP = 50176
NC = 2
NS = 16
NW = NC * NS
B = 128
NB = 200
NB0 = 264
NB1 = 2 * NB - NB0
KB = 8
RB = 4
TOTB = NW * NB
EP = TOTB * B
RPT = NP // NS


def _sc_spmm(src3, dst3, zeros32, tables):
    C = len(tables)
    mesh = plsc.VectorSubcoreMesh(core_axis_name="c", subcore_axis_name="s")

    def body(src_ref, dst_ref, z_ref, *rest):
        table_refs = rest[:C]
        out_ref = rest[C]
        acc, idx_s, idx_d, rows = rest[C + 1:C + 5]
        gsems = rest[C + 5:C + 5 + RB]
        isem = rest[C + 5 + RB]
        cid = lax.axis_index("c")
        sid = lax.axis_index("s")
        base = sid * RPT
        b0 = jnp.where(cid == 0, sid * NB0, NS * NB0 + sid * NB1)
        nsup = jnp.where(cid == 0, NB0 // KB, NB1 // KB)

        def zero_mine():
            pltpu.sync_copy(z_ref.at[pl.ds(base, RPT)],
                            acc.at[pl.ds(base, RPT)])

        def idx_fetch(i, sl):
            pltpu.async_copy(src_ref.at[pl.ds(b0 + i * KB, KB)],
                             idx_s.at[sl], isem)
            pltpu.async_copy(dst_ref.at[pl.ds(b0 + i * KB, KB)],
                             idx_d.at[sl], isem)

        zero_mine()
        plsc.subcore_barrier()
        for c in range(C):
            t_ref = table_refs[c]
            idx_fetch(0, 0)

            def sup(i, carry):
                sl = lax.rem(i, 2)
                pltpu.make_async_copy(src_ref.at[pl.ds(b0 + i * KB, KB)],
                                      idx_s.at[sl], isem).wait()
                pltpu.make_async_copy(dst_ref.at[pl.ds(b0 + i * KB, KB)],
                                      idx_d.at[sl], isem).wait()

                @pl.when(i + 1 < nsup)
                def _():
                    idx_fetch(i + 1, 1 - sl)

                for j in range(RB):
                    pltpu.async_copy(t_ref.at[idx_s.at[sl, j]], rows.at[j],
                                     gsems[j])
                for j in range(KB):
                    q = j % RB
                    pltpu.make_async_copy(t_ref.at[idx_s.at[sl, j]],
                                          rows.at[q], gsems[q]).wait()
                    pltpu.sync_copy(rows.at[q], acc.at[idx_d.at[sl, j]],
                                    add=True)
                    if j + RB < KB:
                        pltpu.async_copy(t_ref.at[idx_s.at[sl, j + RB]],
                                         rows.at[q], gsems[q])
                return carry

            lax.fori_loop(0, nsup, sup, 0)
            plsc.subcore_barrier()
            pltpu.sync_copy(acc.at[pl.ds(base, RPT)],
                            out_ref.at[cid, c, pl.ds(base, RPT)])
            if c < C - 1:
                zero_mine()
                plsc.subcore_barrier()

    f = pl.kernel(
        body,
        out_type=jax.ShapeDtypeStruct((NC, C, NP, 32), jnp.float32),
        mesh=mesh,
        compiler_params=pltpu.CompilerParams(use_tc_tiling_on_sc=False),
        scratch_types=[
            pltpu.VMEM_SHARED((NP, 32), jnp.float32),
            pltpu.VMEM((2, KB, B), jnp.int32),
            pltpu.VMEM((2, KB, B), jnp.int32),
            pltpu.VMEM((RB, B, 32), jnp.float32),
        ] + [pltpu.SemaphoreType.DMA] * (RB + 1),
    )
    return f(src3, dst3, zeros32, *tables)


def _sc_hist(dst3, zeros16, ones16):
    mesh = plsc.VectorSubcoreMesh(core_axis_name="c", subcore_axis_name="s")

    def body(dst_ref, z_ref, ones_ref, out_ref, acc, idx_d, ones_v, sem):
        cid = lax.axis_index("c")
        sid = lax.axis_index("s")
        w = cid * NS + sid
        base = sid * RPT
        pltpu.sync_copy(ones_ref, ones_v)
        pltpu.sync_copy(z_ref.at[pl.ds(base, RPT)], acc.at[pl.ds(base, RPT)])
        plsc.subcore_barrier()

        def sup(i, carry):
            pltpu.sync_copy(dst_ref.at[pl.ds(w * NB + i * KB, KB)], idx_d)
            for j in range(KB):
                pltpu.sync_copy(ones_v, acc.at[idx_d.at[j]], add=True)
            return carry

        lax.fori_loop(0, NB // KB, sup, 0)
        plsc.subcore_barrier()
        pltpu.sync_copy(acc.at[pl.ds(base, RPT)],
                        out_ref.at[cid, pl.ds(base, RPT)])

    f = pl.kernel(
        body,
        out_type=jax.ShapeDtypeStruct((NC, NP, 16), jnp.float32),
        mesh=mesh,
        compiler_params=pltpu.CompilerParams(use_tc_tiling_on_sc=False),
        scratch_types=[
            pltpu.VMEM_SHARED((NP, 16), jnp.float32),
            pltpu.VMEM((KB, B), jnp.int32),
            pltpu.VMEM((B, 16), jnp.float32),
            pltpu.SemaphoreType.DMA,
        ],
    )
    return f(dst3, zeros16, ones16)


GRID = 32
BN = NP // GRID

_node = lambda d: pl.BlockSpec((BN, d), lambda i: (i, 0))
_hist = pl.BlockSpec((NC, BN, 16), lambda i: (0, i, 0))
_part = lambda c: pl.BlockSpec((NC, c, BN, 32), lambda i: (0, 0, i, 0))
_full = lambda a, b: pl.BlockSpec((a, b), lambda i: (0, 0))


def _cnt_dinv(hp_ref):
    cnt = hp_ref[0][:, 0:1] + hp_ref[1][:, 0:1]
    return jnp.maximum(cnt, 1.0), lax.rsqrt(cnt + 1.0)


def _tc1(hp, x8, w1p):
    def body(hp_ref, x_ref, w_ref, o_ref):
        _, dinv = _cnt_dinv(hp_ref)
        o_ref[...] = jnp.dot(x_ref[...], w_ref[...],
                             preferred_element_type=jnp.float32) * dinv

    return pl.pallas_call(
        body, grid=(GRID,),
        in_specs=[_hist, _node(8), _full(8, 32)],
        out_specs=_node(32),
        out_shape=jax.ShapeDtypeStruct((NP, 32), jnp.float32),
    )(hp, x8, w1p)


def _tc2(p1, hp, g1, b1r):
    def body(p_ref, hp_ref, g_ref, b_ref, o_ref):
        _, dinv = _cnt_dinv(hp_ref)
        s = p_ref[0, 0] + p_ref[1, 0] + g_ref[...]
        o_ref[...] = jnp.maximum(s * dinv + b_ref[...], 0.0)

    return pl.pallas_call(
        body, grid=(GRID,),
        in_specs=[_part(1), _hist, _node(32), _full(1, 32)],
        out_specs=_node(32),
        out_shape=jax.ShapeDtypeStruct((NP, 32), jnp.float32),
    )(p1, hp, g1, b1r)


def _tc3(p2, hp, h1, w2l, b2r, w2r):
    def body(p_ref, hp_ref, h_ref, wl_ref, b_ref, wr_ref, *o_refs):
        cnt1, _ = _cnt_dinv(hp_ref)
        agg = (p_ref[0, 0] + p_ref[1, 0]) / cnt1
        h2 = jnp.dot(agg, wl_ref[...], preferred_element_type=jnp.float32)
        h2 = h2 + b_ref[...] + jnp.dot(h_ref[...], wr_ref[...],
                                       preferred_element_type=jnp.float32)
        h2 = jnp.maximum(h2, 0.0)
        for c in range(4):
            o_refs[c][...] = h2[:, 32 * c:32 * (c + 1)]

    shp = jax.ShapeDtypeStruct((NP, 32), jnp.float32)
    return pl.pallas_call(
        body, grid=(GRID,),
        in_specs=[_part(1), _hist, _node(32), _full(32, 128), _full(1, 128),
                  _full(32, 128)],
        out_specs=[_node(32)] * 4,
        out_shape=[shp] * 4,
    )(p2, hp, h1, w2l, b2r, w2r)


def _tc4(p3, hp, h2c, w3l, b3r, w3r, w4l):
    def body(p_ref, hp_ref, hc0, hc1, hc2, hc3, wl_ref, b_ref, wr_ref,
             w4_ref, *o_refs):
        cnt1, _ = _cnt_dinv(hp_ref)
        hcs = (hc0, hc1, hc2, hc3)
        h3 = b_ref[...]
        for c in range(4):
            agg = (p_ref[0, c] + p_ref[1, c]) / cnt1
            h3 = h3 + jnp.dot(agg, wl_ref[32 * c:32 * (c + 1), :],
                              preferred_element_type=jnp.float32)
            h3 = h3 + jnp.dot(hcs[c][...], wr_ref[32 * c:32 * (c + 1), :],
                              preferred_element_type=jnp.float32)
        h3 = jnp.maximum(h3, 0.0)
        for c in range(4):
            o_refs[c][...] = h3[:, 32 * c:32 * (c + 1)]
        o_refs[4][...] = jnp.dot(h3, w4_ref[...],
                                 preferred_element_type=jnp.float32)

    shp = jax.ShapeDtypeStruct((NP, 32), jnp.float32)
    return pl.pallas_call(
        body, grid=(GRID,),
        in_specs=[_part(4), _hist] + [_node(32)] * 4 +
                 [_full(128, 128), _full(1, 128), _full(128, 128),
                  _full(128, 32)],
        out_specs=[_node(32)] * 5,
        out_shape=[shp] * 5,
    )(p3, hp, *h2c, w3l, b3r, w3r, w4l)


def _tc5(p4, hp, h3c, w4r, b4r, wm1, bm1r, wm2p, bm2p):
    def body(p_ref, hp_ref, hc0, hc1, hc2, hc3, wr_ref, b_ref, w1_ref,
             b1_ref, w2_ref, b2_ref, o_ref):
        cnt1, _ = _cnt_dinv(hp_ref)
        hcs = (hc0, hc1, hc2, hc3)
        h4 = (p_ref[0, 0] + p_ref[1, 0]) / cnt1 + b_ref[...]
        for c in range(4):
            h4 = h4 + jnp.dot(hcs[c][...], wr_ref[32 * c:32 * (c + 1), :],
                              preferred_element_type=jnp.float32)
        m = jnp.maximum(jnp.dot(h4, w1_ref[...],
                                preferred_element_type=jnp.float32)
                        + b1_ref[...], 0.0)
        y = jnp.dot(m, w2_ref[...], preferred_element_type=jnp.float32)
        o_ref[...] = jax.nn.sigmoid(y + b2_ref[...])

    return pl.pallas_call(
        body, grid=(GRID,),
        in_specs=[_part(1), _hist] + [_node(32)] * 4 +
                 [_full(128, 32), _full(1, 32), _full(32, 16), _full(1, 16),
                  _full(16, 8), _full(1, 8)],
        out_specs=_node(8),
        out_shape=jax.ShapeDtypeStruct((NP, 8), jnp.float32),
    )(p4, hp, *h3c, w4r, b4r, wm1, bm1r, wm2p, bm2p)


def kernel(x, edge_index, W1, b1, W2l, b2, W2r, W3l, b3, W3r, W4l, b4, W4r,
           Wm1, bm1, Wm2, bm2):
    src = jnp.concatenate([edge_index[0],
                           jnp.zeros((EP - E,), jnp.int32)]).reshape(TOTB, B)
    dst = jnp.concatenate([edge_index[1],
                           jnp.full((EP - E,), N, jnp.int32)]).reshape(TOTB, B)
    zeros32 = jnp.zeros((NP, 32), jnp.float32)
    zeros16 = jnp.zeros((NP, 16), jnp.float32)
    ones16 = jnp.ones((B, 16), jnp.float32)
    x8 = jnp.pad(x, ((0, NP - N), (0, 8 - x.shape[1])))
    w1p = jnp.pad(W1, ((0, 8 - W1.shape[0]), (0, 0)))
    wm2p = jnp.pad(Wm2, ((0, 0), (0, 8 - Wm2.shape[1])))
    bm2p = jnp.pad(bm2, (0, 8 - bm2.shape[0])).reshape(1, 8)
    b1r, b2r, b3r, b4r = (b.reshape(1, -1) for b in (b1, b2, b3, b4))
    bm1r = bm1.reshape(1, -1)

    hp = _sc_hist(dst, zeros16, ones16)
    g1 = _tc1(hp, x8, w1p)
    p1 = _sc_spmm(src, dst, zeros32, [g1])
    h1 = _tc2(p1, hp, g1, b1r)
    p2 = _sc_spmm(src, dst, zeros32, [h1])
    h2c = _tc3(p2, hp, h1, W2l, b2r, W2r)
    p3 = _sc_spmm(src, dst, zeros32, list(h2c))
    *h3c, g4 = _tc4(p3, hp, h2c, W3l, b3r, W3r, W4l)
    p4 = _sc_spmm(src, dst, zeros32, [g4])
    y = _tc5(p4, hp, h3c, W4r, b4r, Wm1, bm1r, wm2p, bm2p)
    return y[:N, 0]
